# baseline (speedup 1.0000x reference)
import functools
import math

import jax
import jax.numpy as jnp
from jax import lax
from jax.experimental import pallas as pl
from jax.experimental.pallas import tpu as pltpu
from jax.experimental.pallas import tpu_sc as plsc

ALPHA = 0.5
_S_EDGE = math.sqrt(ALPHA)
_S_LOOP = math.sqrt(1.0 - ALPHA)


def _prep_body(n, feat_ref, w_ref, et_ref, src_ref, table_ref, g_ref):
    x = feat_ref[...]
    r_count = w_ref.shape[0]
    for r in range(r_count):
        table_ref[r] = (
            jnp.dot(x, w_ref[r], preferred_element_type=jnp.float32) * _S_EDGE
        )

    @pl.when(pl.program_id(0) == 0)
    def _():
        g_ref[...] = et_ref[...] * n + src_ref[...]


def _prepare(feat, weight, etypes, src, blk):
    n, d_in = feat.shape
    r_count, _, d_out = weight.shape
    e = etypes.shape[0]
    grid = n // blk
    erows = e // 128
    et2 = etypes.reshape(erows, 128)
    src2 = src.reshape(erows, 128)
    table, g2 = pl.pallas_call(
        functools.partial(_prep_body, n),
        grid=(grid,),
        in_specs=[
            pl.BlockSpec((blk, d_in), lambda i: (i, 0)),
            pl.BlockSpec((r_count, d_in, d_out), lambda i: (0, 0, 0)),
            pl.BlockSpec((erows, 128), lambda i: (0, 0)),
            pl.BlockSpec((erows, 128), lambda i: (0, 0)),
        ],
        out_specs=(
            pl.BlockSpec((r_count, blk, d_out), lambda i: (0, i, 0)),
            pl.BlockSpec((erows, 128), lambda i: (0, 0)),
        ),
        out_shape=(
            jax.ShapeDtypeStruct((r_count, n, d_out), jnp.float32),
            jax.ShapeDtypeStruct((erows, 128), jnp.int32),
        ),
    )(feat, weight, et2, src2)
    return table, g2.reshape(e)


def _make_sc_kernel(n, e, d, nc, ns, ch, zrows, nlanes=4):
    nw = nc * ns
    ep = e // nw
    nchunk = ep // ch
    ring = 2 * nlanes
    pre = nchunk % nlanes
    rows_base = (n // (8 * ns)) * 8
    tail = n - ns * rows_base
    mesh = plsc.VectorSubcoreMesh(core_axis_name="c", subcore_axis_name="s")

    @functools.partial(
        pl.kernel,
        out_type=jax.ShapeDtypeStruct((nc, n, d), jnp.float32),
        mesh=mesh,
        scratch_types=(
            [
                pltpu.VMEM((ring, ch), jnp.int32),
                pltpu.VMEM((ring, ch), jnp.int32),
            ]
            + [pltpu.VMEM((ch, d), jnp.float32) for _ in range(nlanes)]
            + [
                pltpu.VMEM((zrows, d), jnp.float32),
                pltpu.VMEM_SHARED((n, d), jnp.float32),
            ]
            + [pltpu.SemaphoreType.DMA] * (3 * nlanes + 1)
        ),
    )
    def sc_fn(g_h, dst_h, table, out, gidx, didx, *rest):
        rows = list(rest[:nlanes])
        zbuf = rest[nlanes]
        acc = rest[nlanes + 1]
        sems = rest[nlanes + 2:]
        lsems = list(sems[:nlanes])
        gsems = list(sems[nlanes:2 * nlanes])
        ssems = list(sems[2 * nlanes:3 * nlanes])
        zsem = sems[3 * nlanes]

        c = lax.axis_index("c")
        s = lax.axis_index("s")
        wid = s * nc + c
        base_e = pl.multiple_of(wid * ep, 8)

        def load_idx(j, lsem):
            slot = lax.rem(j, ring)
            off = pl.multiple_of(base_e + j * ch, 8)
            pltpu.async_copy(g_h.at[pl.ds(off, ch)], gidx.at[slot], lsem)
            pltpu.async_copy(dst_h.at[pl.ds(off, ch)], didx.at[slot], lsem)

        def wait_idx(lsem):
            pltpu.make_async_copy(
                g_h.at[pl.ds(0, ch)], gidx.at[0], lsem).wait()
            pltpu.make_async_copy(
                dst_h.at[pl.ds(0, ch)], didx.at[0], lsem).wait()

        def gather(j, rows_l, sem):
            pltpu.async_copy(table.at[gidx.at[lax.rem(j, ring)]], rows_l, sem)

        def wait_gather(rows_l, sem):
            pltpu.make_async_copy(table.at[gidx.at[0]], rows_l, sem).wait()

        def scat(j, rows_l, sem):
            pltpu.async_copy(
                rows_l, acc.at[didx.at[lax.rem(j, ring)]], sem, add=True)

        def wait_scat(rows_l, sem):
            pltpu.make_async_copy(rows_l, acc.at[didx.at[0]], sem).wait()

        for j in range(pre):
            load_idx(j, lsems[j])

        def zero_row(i, carry):
            for k in range(d // 16):
                zbuf[i, pl.ds(k * 16, 16)] = jnp.zeros((16,), jnp.float32)
            return carry

        lax.fori_loop(0, zrows, zero_row, 0)
        r0 = s * rows_base
        for k in range(rows_base // zrows):
            pltpu.async_copy(zbuf, acc.at[pl.ds(r0 + k * zrows, zrows)], zsem)
        if tail:
            @pl.when(s == ns - 1)
            def _init_tail():
                pltpu.async_copy(
                    zbuf.at[pl.ds(0, tail)],
                    acc.at[pl.ds(ns * rows_base, tail)],
                    zsem,
                )
        for k in range(rows_base // zrows):
            pltpu.make_async_copy(
                zbuf, acc.at[pl.ds(r0 + k * zrows, zrows)], zsem).wait()
        if tail:
            @pl.when(s == ns - 1)
            def _drain_tail():
                pltpu.make_async_copy(
                    zbuf.at[pl.ds(0, tail)],
                    acc.at[pl.ds(ns * rows_base, tail)],
                    zsem,
                ).wait()
        plsc.subcore_barrier()

        for j in range(pre):
            wait_idx(lsems[j])
            gather(j, rows[j], gsems[j])
            wait_gather(rows[j], gsems[j])
            scat(j, rows[j], ssems[j])
        for lane in range(nlanes):
            load_idx(pre + lane, lsems[lane])

        def body(i, carry):
            chunks = [pre + nlanes * i + lane for lane in range(nlanes)]
            for lane in range(nlanes):
                x = chunks[lane]
                wait_idx(lsems[lane])
                if lane < pre:
                    wait_scat(rows[lane], ssems[lane])
                else:
                    @pl.when(i > 0)
                    def _(lane=lane):
                        wait_scat(rows[lane], ssems[lane])
                gather(x, rows[lane], gsems[lane])
            for lane in range(nlanes):
                x = chunks[lane]

                @pl.when(x + nlanes < nchunk)
                def _(lane=lane, x=x):
                    load_idx(x + nlanes, lsems[lane])
            for lane in range(nlanes):
                x = chunks[lane]
                wait_gather(rows[lane], gsems[lane])
                scat(x, rows[lane], ssems[lane])
            return carry

        lax.fori_loop(0, (nchunk - pre) // nlanes, body, 0)
        for lane in range(nlanes):
            wait_scat(rows[lane], ssems[lane])
        plsc.subcore_barrier()

        pltpu.sync_copy(
            acc.at[pl.ds(r0, rows_base)],
            out.at[c, pl.ds(r0, rows_base)],
        )
        if tail:
            @pl.when(s == ns - 1)
            def _out_tail():
                pltpu.sync_copy(
                    acc.at[pl.ds(ns * rows_base, tail)],
                    out.at[c, pl.ds(ns * rows_base, tail)],
                )

    return sc_fn


def _combine_body(p_ref, feat_ref, lw_ref, b_ref, out_ref):
    loop_msg = jnp.dot(
        feat_ref[...], lw_ref[...], preferred_element_type=jnp.float32
    )
    acc = p_ref[0]
    for c in range(1, p_ref.shape[0]):
        acc = acc + p_ref[c]
    out_ref[...] = acc + loop_msg * _S_LOOP + b_ref[...]


def _combine(partials, feat, loop_weight, h_bias, blk):
    n, d_in = feat.shape
    d_out = loop_weight.shape[1]
    nc = partials.shape[0]
    grid = n // blk
    return pl.pallas_call(
        _combine_body,
        grid=(grid,),
        in_specs=[
            pl.BlockSpec((nc, blk, d_out), lambda i: (0, i, 0)),
            pl.BlockSpec((blk, d_in), lambda i: (i, 0)),
            pl.BlockSpec((d_in, d_out), lambda i: (0, 0)),
            pl.BlockSpec((1, d_out), lambda i: (0, 0)),
        ],
        out_specs=pl.BlockSpec((blk, d_out), lambda i: (i, 0)),
        out_shape=jax.ShapeDtypeStruct((n, d_out), jnp.float32),
    )(partials, feat, loop_weight, h_bias.reshape(1, d_out))


def kernel(feat, edge_index, etypes, weight, h_bias, loop_weight):
    n, d_in = feat.shape
    r_count, _, d_out = weight.shape
    e = etypes.shape[0]

    info = plsc.get_sparse_core_info()
    nc, ns = info.num_cores, info.num_subcores

    src = edge_index[0]
    dst = edge_index[1]
    table, g = _prepare(feat, weight, etypes, src, blk=1000)
    table2d = table.reshape(r_count * n, d_out)

    sc_fn = _make_sc_kernel(n, e, d_out, nc, ns, ch=16, zrows=24, nlanes=8)
    partials = sc_fn(g, dst, table2d)

    return _combine(partials, feat, loop_weight, h_bias, blk=1000)


# speedup vs baseline: 1.2786x; 1.2786x over previous
import functools
import math

import jax
import jax.numpy as jnp
from jax import lax
from jax.experimental import pallas as pl
from jax.experimental.pallas import tpu as pltpu
from jax.experimental.pallas import tpu_sc as plsc

ALPHA = 0.5
_S_EDGE = math.sqrt(ALPHA)
_S_LOOP = math.sqrt(1.0 - ALPHA)


def _prep_body(n, feat_ref, w_ref, et_ref, src_ref, table_ref, g_ref):
    x = feat_ref[...]
    r_count = w_ref.shape[0]
    for r in range(r_count):
        table_ref[r] = (
            jnp.dot(x, w_ref[r], preferred_element_type=jnp.float32) * _S_EDGE
        )

    @pl.when(pl.program_id(0) == 0)
    def _():
        g_ref[...] = et_ref[...] * n + src_ref[...]


def _prepare(feat, weight, etypes, src, blk):
    n, d_in = feat.shape
    r_count, _, d_out = weight.shape
    e = etypes.shape[0]
    grid = n // blk
    erows = e // 128
    et2 = etypes.reshape(erows, 128)
    src2 = src.reshape(erows, 128)
    table, g2 = pl.pallas_call(
        functools.partial(_prep_body, n),
        grid=(grid,),
        in_specs=[
            pl.BlockSpec((blk, d_in), lambda i: (i, 0)),
            pl.BlockSpec((r_count, d_in, d_out), lambda i: (0, 0, 0)),
            pl.BlockSpec((erows, 128), lambda i: (0, 0)),
            pl.BlockSpec((erows, 128), lambda i: (0, 0)),
        ],
        out_specs=(
            pl.BlockSpec((r_count, blk, d_out), lambda i: (0, i, 0)),
            pl.BlockSpec((erows, 128), lambda i: (0, 0)),
        ),
        out_shape=(
            jax.ShapeDtypeStruct((r_count, n, d_out), jnp.float32),
            jax.ShapeDtypeStruct((erows, 128), jnp.int32),
        ),
    )(feat, weight, et2, src2)
    return table, g2.reshape(e)


def _make_sc_kernel(n, e, d, nc, ns, ch, zrows, nlanes=4):
    nw = nc * ns
    ep = e // nw
    nchunk = ep // ch
    ring = 2 * nlanes
    pre = nchunk % nlanes
    rows_base = (n // (8 * ns)) * 8
    tail = n - ns * rows_base
    mesh = plsc.VectorSubcoreMesh(core_axis_name="c", subcore_axis_name="s")

    @functools.partial(
        pl.kernel,
        out_type=jax.ShapeDtypeStruct((nc, n, d), jnp.float32),
        mesh=mesh,
        scratch_types=(
            [
                pltpu.VMEM((ring, ch), jnp.int32),
                pltpu.VMEM((ring, ch), jnp.int32),
            ]
            + [pltpu.VMEM((ch, d), jnp.float32) for _ in range(nlanes)]
            + [
                pltpu.VMEM((zrows, d), jnp.float32),
                pltpu.VMEM_SHARED((n, d), jnp.float32),
            ]
            + [pltpu.SemaphoreType.DMA] * (3 * nlanes + 1)
        ),
    )
    def sc_fn(g_h, dst_h, table, out, gidx, didx, *rest):
        rows = list(rest[:nlanes])
        zbuf = rest[nlanes]
        acc = rest[nlanes + 1]
        sems = rest[nlanes + 2:]
        lsems = list(sems[:nlanes])
        gsems = list(sems[nlanes:2 * nlanes])
        ssems = list(sems[2 * nlanes:3 * nlanes])
        zsem = sems[3 * nlanes]

        c = lax.axis_index("c")
        s = lax.axis_index("s")
        wid = s * nc + c
        base_e = pl.multiple_of(wid * ep, 8)

        def load_idx(j, lsem):
            slot = lax.rem(j, ring)
            off = pl.multiple_of(base_e + j * ch, 8)
            pltpu.async_copy(g_h.at[pl.ds(off, ch)], gidx.at[slot], lsem)
            pltpu.async_copy(dst_h.at[pl.ds(off, ch)], didx.at[slot], lsem)

        def wait_idx(lsem):
            pltpu.make_async_copy(
                g_h.at[pl.ds(0, ch)], gidx.at[0], lsem).wait()
            pltpu.make_async_copy(
                dst_h.at[pl.ds(0, ch)], didx.at[0], lsem).wait()

        def gather(j, rows_l, sem):
            pltpu.async_copy(table.at[gidx.at[lax.rem(j, ring)]], rows_l, sem)

        def wait_gather(rows_l, sem):
            pltpu.make_async_copy(table.at[gidx.at[0]], rows_l, sem).wait()

        def scat(j, rows_l, sem):
            pltpu.async_copy(
                rows_l, acc.at[didx.at[lax.rem(j, ring)]], sem, add=True)

        def wait_scat(rows_l, sem):
            pltpu.make_async_copy(rows_l, acc.at[didx.at[0]], sem).wait()

        for j in range(pre):
            load_idx(j, lsems[j])

        def zero_row(i, carry):
            for k in range(d // 16):
                zbuf[i, pl.ds(k * 16, 16)] = jnp.zeros((16,), jnp.float32)
            return carry

        lax.fori_loop(0, zrows, zero_row, 0)
        r0 = s * rows_base
        for k in range(rows_base // zrows):
            pltpu.async_copy(zbuf, acc.at[pl.ds(r0 + k * zrows, zrows)], zsem)
        if tail:
            @pl.when(s == ns - 1)
            def _init_tail():
                pltpu.async_copy(
                    zbuf.at[pl.ds(0, tail)],
                    acc.at[pl.ds(ns * rows_base, tail)],
                    zsem,
                )
        for j in range(pre):
            wait_idx(lsems[j])
            gather(j, rows[j], gsems[j])
        for lane in range(nlanes):
            load_idx(pre + lane, lsems[lane])
        for k in range(rows_base // zrows):
            pltpu.make_async_copy(
                zbuf, acc.at[pl.ds(r0 + k * zrows, zrows)], zsem).wait()
        if tail:
            @pl.when(s == ns - 1)
            def _drain_tail():
                pltpu.make_async_copy(
                    zbuf.at[pl.ds(0, tail)],
                    acc.at[pl.ds(ns * rows_base, tail)],
                    zsem,
                ).wait()
        plsc.subcore_barrier()
        for j in range(pre):
            wait_gather(rows[j], gsems[j])
            scat(j, rows[j], ssems[j])

        def body(i, carry):
            chunks = [pre + nlanes * i + lane for lane in range(nlanes)]
            for lane in range(nlanes):
                x = chunks[lane]
                wait_idx(lsems[lane])
                if lane < pre:
                    wait_scat(rows[lane], ssems[lane])
                else:
                    @pl.when(i > 0)
                    def _(lane=lane):
                        wait_scat(rows[lane], ssems[lane])
                gather(x, rows[lane], gsems[lane])
            for lane in range(nlanes):
                x = chunks[lane]

                @pl.when(x + nlanes < nchunk)
                def _(lane=lane, x=x):
                    load_idx(x + nlanes, lsems[lane])
            for lane in range(nlanes):
                x = chunks[lane]
                wait_gather(rows[lane], gsems[lane])
                scat(x, rows[lane], ssems[lane])
            return carry

        lax.fori_loop(0, (nchunk - pre) // nlanes, body, 0)
        for lane in range(nlanes):
            wait_scat(rows[lane], ssems[lane])
        plsc.subcore_barrier()

        pltpu.sync_copy(
            acc.at[pl.ds(r0, rows_base)],
            out.at[c, pl.ds(r0, rows_base)],
        )
        if tail:
            @pl.when(s == ns - 1)
            def _out_tail():
                pltpu.sync_copy(
                    acc.at[pl.ds(ns * rows_base, tail)],
                    out.at[c, pl.ds(ns * rows_base, tail)],
                )

    return sc_fn


def _combine_body(p_ref, feat_ref, lw_ref, b_ref, out_ref):
    loop_msg = jnp.dot(
        feat_ref[...], lw_ref[...], preferred_element_type=jnp.float32
    )
    acc = p_ref[0]
    for c in range(1, p_ref.shape[0]):
        acc = acc + p_ref[c]
    out_ref[...] = acc + loop_msg * _S_LOOP + b_ref[...]


def _combine(partials, feat, loop_weight, h_bias, blk):
    n, d_in = feat.shape
    d_out = loop_weight.shape[1]
    nc = partials.shape[0]
    grid = n // blk
    return pl.pallas_call(
        _combine_body,
        grid=(grid,),
        in_specs=[
            pl.BlockSpec((nc, blk, d_out), lambda i: (0, i, 0)),
            pl.BlockSpec((blk, d_in), lambda i: (i, 0)),
            pl.BlockSpec((d_in, d_out), lambda i: (0, 0)),
            pl.BlockSpec((1, d_out), lambda i: (0, 0)),
        ],
        out_specs=pl.BlockSpec((blk, d_out), lambda i: (i, 0)),
        out_shape=jax.ShapeDtypeStruct((n, d_out), jnp.float32),
    )(partials, feat, loop_weight, h_bias.reshape(1, d_out))


def kernel(feat, edge_index, etypes, weight, h_bias, loop_weight):
    n, d_in = feat.shape
    r_count, _, d_out = weight.shape
    e = etypes.shape[0]

    info = plsc.get_sparse_core_info()
    nc, ns = info.num_cores, info.num_subcores

    src = edge_index[0]
    dst = edge_index[1]
    table, g = _prepare(feat, weight, etypes, src, blk=1000)
    table2d = table.reshape(r_count * n, d_out)

    sc_fn = _make_sc_kernel(n, e, d_out, nc, ns, ch=40, zrows=24, nlanes=6)
    partials = sc_fn(g, dst, table2d)

    return _combine(partials, feat, loop_weight, h_bias, blk=1000)
